# Initial kernel scaffold; baseline (speedup 1.0000x reference)
import jax
import jax.numpy as jnp
from jax.experimental import pallas as pl


def kernel(Y, X, x, y):
    raise NotImplementedError("write your pallas kernel here")


# speedup vs baseline: 1.9406x; 1.9406x over previous
import jax
import jax.numpy as jnp
from jax import lax
from jax.experimental import pallas as pl
from jax.experimental.pallas import tpu as pltpu

B, H, W, NSRC = 8, 2048, 2048, 128
BH = 256
NBH = H // BH


def _body(lo_ref, hi_ref, x_ref, y_ref, xv_ref, yin, yout):
    g = pl.program_id(1)
    yout[...] = yin[...]
    r0 = g * BH

    def upd(i, carry):
        dx = x_ref[i] - r0
        yi = y_ref[i]
        xv = xv_ref[i]
        col = lax.broadcasted_iota(jnp.int32, (1, 1, W), 2)
        row = yout[:, pl.ds(dx, 1), :]
        yout[:, pl.ds(dx, 1), :] = row + jnp.where(col == yi, xv, 0.0)
        return carry

    lax.fori_loop(lo_ref[g], hi_ref[g], upd, 0)


def kernel(Y, X, x, y):
    block_starts = jnp.arange(NBH, dtype=jnp.int32) * BH
    lo = jnp.searchsorted(x, block_starts, side="left").astype(jnp.int32)
    hi = jnp.searchsorted(x, block_starts + BH, side="left").astype(jnp.int32)

    grid_spec = pltpu.PrefetchScalarGridSpec(
        num_scalar_prefetch=5,
        grid=(B, NBH),
        in_specs=[
            pl.BlockSpec((1, BH, W), lambda b, g, *refs: (b, g, 0)),
        ],
        out_specs=pl.BlockSpec((1, BH, W), lambda b, g, *refs: (b, g, 0)),
    )
    return pl.pallas_call(
        _body,
        grid_spec=grid_spec,
        out_shape=jax.ShapeDtypeStruct((B, H, W), jnp.float32),
    )(lo, hi, x, y, X, Y)


# speedup vs baseline: 2.0169x; 1.0393x over previous
import jax
import jax.numpy as jnp
from jax import lax
from jax.experimental import pallas as pl
from jax.experimental.pallas import tpu as pltpu

B, H, W, NSRC = 8, 2048, 2048, 128
FR = 1024
NBLK = (B * H) // FR


def _body(lo_ref, hi_ref, xf_ref, yf_ref, xvf_ref, yin, yout):
    g = pl.program_id(0)
    yout[...] = yin[...]
    r0 = g * FR

    def upd(i, carry):
        dr = xf_ref[i] - r0
        yi = yf_ref[i]
        xv = xvf_ref[i]
        col = lax.broadcasted_iota(jnp.int32, (1, W), 1)
        row = yout[pl.ds(dr, 1), :]
        yout[pl.ds(dr, 1), :] = row + jnp.where(col == yi, xv, 0.0)
        return carry

    lax.fori_loop(lo_ref[g], hi_ref[g], upd, 0)


def kernel(Y, X, x, y):
    Yf = Y.reshape(B * H, W)
    xf = (jnp.arange(B, dtype=jnp.int32)[:, None] * H + x[None, :]).reshape(-1)
    yf = jnp.broadcast_to(y, (B, NSRC)).reshape(-1)
    xvf = jnp.broadcast_to(X, (B, NSRC)).reshape(-1)

    block_starts = jnp.arange(NBLK, dtype=jnp.int32) * FR
    lo = jnp.searchsorted(xf, block_starts, side="left").astype(jnp.int32)
    hi = jnp.searchsorted(xf, block_starts + FR, side="left").astype(jnp.int32)

    grid_spec = pltpu.PrefetchScalarGridSpec(
        num_scalar_prefetch=5,
        grid=(NBLK,),
        in_specs=[
            pl.BlockSpec((FR, W), lambda g, *refs: (g, 0)),
        ],
        out_specs=pl.BlockSpec((FR, W), lambda g, *refs: (g, 0)),
    )
    out = pl.pallas_call(
        _body,
        grid_spec=grid_spec,
        out_shape=jax.ShapeDtypeStruct((B * H, W), jnp.float32),
    )(lo, hi, xf, yf, xvf, Yf)
    return out.reshape(B, H, W)
